# baseline (speedup 1.0000x reference)
import jax
import jax.numpy as jnp
from jax.experimental import pallas as pl
from jax.experimental.pallas import tpu as pltpu


def _conv_body(L, LP, E, F, K, U, xf_ref, tbl_ref, wt_ref, b_ref,
               hp_ref, hpT_ref, tbl_v, emb_s, sem):
    b = pl.program_id(0)
    half = K // 2

    @pl.when(b == 0)
    def _():
        cp = pltpu.make_async_copy(tbl_ref, tbl_v, sem)
        cp.start()
        cp.wait()

    emb_s[0:half, 0, :] = jnp.zeros((half, E), jnp.float32)
    nz = emb_s.shape[0] - half - L
    emb_s[half + L:, 0, :] = jnp.zeros((nz, E), jnp.float32)

    base = b * L

    def gather_chunk(o, carry):
        s = o * U
        for u in range(U):
            idx = xf_ref[base + s + u]
            emb_s[pl.ds(half + s + u, 1)] = tbl_v[pl.ds(idx, 1)]
        return carry

    jax.lax.fori_loop(0, L // U, gather_chunk, 0)

    e = emb_s[:, 0, :]
    acc = jnp.zeros((LP, F), jnp.float32)
    for k in range(K):
        acc = acc + jnp.dot(e[k:k + LP, :], wt_ref[k],
                            preferred_element_type=jnp.float32)
    h = jnp.tanh(acc + b_ref[...])
    rows = jax.lax.broadcasted_iota(jnp.int32, (LP, F), 0)
    h = jnp.where(rows < L, h, 0.0).astype(jnp.bfloat16)
    hp_ref[0] = h
    ones = jnp.ones((1, h.shape[0]), jnp.bfloat16)
    hpT_ref[0] = jnp.concatenate([h.T, ones], axis=0)


def _attn_body(B, L, F, YB, G,
               hp_ref, hpT_ref, u4T_ref, fw_ref, gcn_w_ref,
               S_ref, y4t_ref, y4a_ref):
    b = pl.program_id(1)
    hp = hp_ref[b][:L]
    hpT1 = hpT_ref[b][:, :L]
    H = YB // G
    sups = []
    for g in range(G):
        sl = slice(g * H, (g + 1) * H)
        sT = jnp.dot(hp, u4T_ref[:, sl],
                     preferred_element_type=jnp.float32)
        e = jnp.exp2(sT).astype(jnp.bfloat16)
        m1 = jnp.dot(hpT1, e,
                     preferred_element_type=jnp.float32)
        m4tT = m1[:F] * (1.0 / m1[F:F + 1])
        y4t_ref[0, :, sl] = (jnp.sum(m4tT * fw_ref[0:F, sl], axis=0,
                                     keepdims=True)
                             + fw_ref[2 * F:2 * F + 1, sl])
        y4a_ref[0, :, sl] = jnp.sum(m4tT * fw_ref[F:2 * F, sl], axis=0,
                                    keepdims=True)
        sup = jax.lax.dot_general(
            m4tT, gcn_w_ref[...], (((0,), (0,)), ((), ())),
            preferred_element_type=jnp.float32)
        sups.append(sup.astype(jnp.bfloat16))
    stripe = jnp.concatenate(sups, axis=0)
    for j in range(B):
        @pl.when(b == j)
        def _():
            S_ref[:, j * F:(j + 1) * F] = stripe


def _gcn_body(B, F, adj_ref, S_ref, wB_ref, gb_ref, sel_ref, y4a_ref,
              f4b_ref, y4_ref):
    out1 = jax.lax.dot_general(
        adj_ref[...], S_ref[...], (((1,), (0,)), ((), ())),
        preferred_element_type=jnp.float32)
    out1 = out1 + jnp.tile(gb_ref[...], (1, B))
    out1 = jnp.where(out1 >= 0.0, out1, 0.2 * out1)
    prod = out1 * jnp.tile(wB_ref[...], (1, B))
    cols = jnp.dot(prod, sel_ref[...],
                   preferred_element_type=jnp.float32)
    y4_ref[...] = y4a_ref[...] + cols.T + f4b_ref[...]


def kernel(x, target, embed_w, conv_w, conv_b, U4_w, gcn_w, gcn_b, adj,
           final4t_w, final4t_b, final4_w, final4_b):
    B, L = x.shape
    V, E = embed_w.shape
    F = conv_w.shape[0]
    K = conv_w.shape[2]
    Y = U4_w.shape[0]
    LP = ((L + 127) // 128) * 128
    YB = 512
    NY = (Y + YB - 1) // YB
    IB = 256
    NI = (Y + IB - 1) // IB
    half = K // 2

    xf = x.astype(jnp.int32).reshape(-1)
    tbl3 = embed_w.reshape(V, 1, E)
    wt = conv_w.transpose(2, 1, 0)
    cb = conv_b.reshape(1, F)
    LOG2E = 1.4426950408889634
    u4T = (U4_w.T * LOG2E).astype(jnp.bfloat16)
    fw = jnp.concatenate([final4t_w.T, final4_w[:, :F].T,
                          final4t_b.reshape(1, Y)], axis=0)
    wB = final4_w[:, F:]
    gb1 = gcn_b.reshape(1, F)
    sel = (jax.lax.broadcasted_iota(jnp.int32, (B * F, B), 0) // F
           == jax.lax.broadcasted_iota(jnp.int32, (B * F, B), 1)
           ).astype(jnp.float32)
    f4b = final4_b.reshape(1, Y)

    from functools import partial
    U = 25
    hp, hpT = pl.pallas_call(
        partial(_conv_body, L, LP, E, F, K, U),
        grid_spec=pltpu.PrefetchScalarGridSpec(
            num_scalar_prefetch=1,
            grid=(B,),
            in_specs=[
                pl.BlockSpec(memory_space=pl.ANY),
                pl.BlockSpec((K, E, F), lambda b, xf: (0, 0, 0)),
                pl.BlockSpec((1, F), lambda b, xf: (0, 0)),
            ],
            out_specs=[
                pl.BlockSpec((1, LP, F), lambda b, xf: (b, 0, 0)),
                pl.BlockSpec((1, F + 1, LP), lambda b, xf: (b, 0, 0)),
            ],
            scratch_shapes=[
                pltpu.VMEM((V, 1, E), jnp.float32),
                pltpu.VMEM((LP + K - 1, 1, E), jnp.float32),
                pltpu.SemaphoreType.DMA,
            ],
        ),
        out_shape=[
            jax.ShapeDtypeStruct((B, LP, F), jnp.bfloat16),
            jax.ShapeDtypeStruct((B, F + 1, LP), jnp.bfloat16),
        ],
        compiler_params=pltpu.CompilerParams(
            dimension_semantics=("arbitrary",),
            vmem_limit_bytes=48 * 1024 * 1024),
        name="conv_tanh",
    )(xf, tbl3, wt, cb)

    S_flat, y4t3, y4a3 = pl.pallas_call(
        partial(_attn_body, B, L, F, YB, 2),
        grid=(NY, B),
        in_specs=[
            pl.BlockSpec((B, LP, F), lambda i, b: (0, 0, 0)),
            pl.BlockSpec((B, F + 1, LP), lambda i, b: (0, 0, 0)),
            pl.BlockSpec((F, YB), lambda i, b: (0, i)),
            pl.BlockSpec((2 * F + 1, YB), lambda i, b: (0, i)),
            pl.BlockSpec((F, F), lambda i, b: (0, 0)),
        ],
        out_specs=[
            pl.BlockSpec((YB, B * F), lambda i, b: (i, 0)),
            pl.BlockSpec((1, 1, YB), lambda i, b: (b, 0, i)),
            pl.BlockSpec((1, 1, YB), lambda i, b: (b, 0, i)),
        ],
        out_shape=[
            jax.ShapeDtypeStruct((Y, B * F), jnp.bfloat16),
            jax.ShapeDtypeStruct((B, 1, Y), jnp.float32),
            jax.ShapeDtypeStruct((B, 1, Y), jnp.float32),
        ],
        compiler_params=pltpu.CompilerParams(
            dimension_semantics=("parallel", "arbitrary"),
            vmem_limit_bytes=48 * 1024 * 1024),
        name="attn_pool",
    )(hp, hpT, u4T, fw, gcn_w)

    y4t = y4t3.reshape(B, Y)
    y4a = y4a3.reshape(B, Y)

    y4 = pl.pallas_call(
        partial(_gcn_body, B, F),
        grid=(NI,),
        in_specs=[
            pl.BlockSpec((IB, Y), lambda i: (i, 0)),
            pl.BlockSpec((Y, B * F), lambda i: (0, 0)),
            pl.BlockSpec((IB, F), lambda i: (i, 0)),
            pl.BlockSpec((1, F), lambda i: (0, 0)),
            pl.BlockSpec((B * F, B), lambda i: (0, 0)),
            pl.BlockSpec((B, IB), lambda i: (0, i)),
            pl.BlockSpec((1, IB), lambda i: (0, i)),
        ],
        out_specs=pl.BlockSpec((B, IB), lambda i: (0, i)),
        out_shape=jax.ShapeDtypeStruct((B, Y), jnp.float32),
        compiler_params=pltpu.CompilerParams(
            dimension_semantics=("parallel",),
            vmem_limit_bytes=56 * 1024 * 1024),
        name="gcn_score",
    )(adj, S_flat, wB, gb1, sel, y4a, f4b)

    return y4t, y4


# speedup vs baseline: 1.2297x; 1.2297x over previous
import jax
import jax.numpy as jnp
from jax.experimental import pallas as pl
from jax.experimental.pallas import tpu as pltpu


def _conv_body(L, LP, E, F, K, U, xf_ref, tbl_ref, wt_ref, b_ref,
               hp_ref, hpT_ref, tbl_v, emb_s, sem):
    b = pl.program_id(0)
    half = K // 2

    @pl.when(b == 0)
    def _():
        cp = pltpu.make_async_copy(tbl_ref, tbl_v, sem)
        cp.start()
        cp.wait()

    emb_s[0:half, 0, :] = jnp.zeros((half, E), jnp.float32)
    nz = emb_s.shape[0] - half - L
    emb_s[half + L:, 0, :] = jnp.zeros((nz, E), jnp.float32)

    base = b * L

    def gather_chunk(o, carry):
        s = o * U
        for u in range(U):
            idx = xf_ref[base + s + u]
            emb_s[pl.ds(half + s + u, 1)] = tbl_v[pl.ds(idx, 1)]
        return carry

    jax.lax.fori_loop(0, L // U, gather_chunk, 0)

    e = emb_s[:, 0, :]
    acc = jnp.zeros((LP, F), jnp.float32)
    for k in range(K):
        acc = acc + jnp.dot(e[k:k + LP, :], wt_ref[k],
                            preferred_element_type=jnp.float32)
    h = jnp.tanh(acc + b_ref[...])
    rows = jax.lax.broadcasted_iota(jnp.int32, (LP, F), 0)
    h = jnp.where(rows < L, h, 0.0).astype(jnp.bfloat16)
    hp_ref[0] = h
    ones = jnp.ones((1, h.shape[0]), jnp.bfloat16)
    hpT_ref[0] = jnp.concatenate([h.T, ones], axis=0)


def _attn_body(B, L, F, YB, G,
               hp_ref, hpT_ref, u4T_ref, fw_ref, gcn_w_ref,
               S_ref, y4t_ref, y4a_ref):
    b = pl.program_id(1)
    hp = hp_ref[b][:L]
    hpT1 = hpT_ref[b][:, :L]
    H = YB // G
    sups = []
    for g in range(G):
        sl = slice(g * H, (g + 1) * H)
        sT = jnp.dot(hp, u4T_ref[:, sl],
                     preferred_element_type=jnp.float32)
        e = jnp.exp2(sT).astype(jnp.bfloat16)
        m1 = jnp.dot(hpT1, e,
                     preferred_element_type=jnp.float32)
        m4tT = m1[:F] * (1.0 / m1[F:F + 1])
        y4t_ref[0, :, sl] = (jnp.sum(m4tT * fw_ref[0:F, sl], axis=0,
                                     keepdims=True)
                             + fw_ref[2 * F:2 * F + 1, sl])
        y4a_ref[0, :, sl] = jnp.sum(m4tT * fw_ref[F:2 * F, sl], axis=0,
                                    keepdims=True)
        sup = jax.lax.dot_general(
            m4tT, gcn_w_ref[...], (((0,), (0,)), ((), ())),
            preferred_element_type=jnp.float32)
        sups.append(sup.astype(jnp.bfloat16))
    stripe = jnp.concatenate(sups, axis=0)
    for j in range(B):
        @pl.when(b == j)
        def _():
            S_ref[:, j * F:(j + 1) * F] = stripe


def _gcn_body(B, F, adj_ref, S_ref, wB_ref, gb_ref, sel_ref, y4a_ref,
              f4b_ref, y4_ref):
    out1 = jax.lax.dot_general(
        adj_ref[...], S_ref[...], (((1,), (0,)), ((), ())),
        preferred_element_type=jnp.float32)
    out1 = out1 + jnp.tile(gb_ref[...], (1, B))
    out1 = jnp.where(out1 >= 0.0, out1, 0.2 * out1)
    prod = out1 * jnp.tile(wB_ref[...], (1, B))
    cols = jnp.dot(prod, sel_ref[...],
                   preferred_element_type=jnp.float32)
    y4_ref[...] = y4a_ref[...] + cols.T + f4b_ref[...]


def kernel(x, target, embed_w, conv_w, conv_b, U4_w, gcn_w, gcn_b, adj,
           final4t_w, final4t_b, final4_w, final4_b):
    B, L = x.shape
    V, E = embed_w.shape
    F = conv_w.shape[0]
    K = conv_w.shape[2]
    Y = U4_w.shape[0]
    LP = ((L + 127) // 128) * 128
    YB = 512
    NY = (Y + YB - 1) // YB
    IB = 256
    NI = (Y + IB - 1) // IB
    half = K // 2

    xf = x.astype(jnp.int32).reshape(-1)
    tbl3 = embed_w.reshape(V, 1, E)
    wt = conv_w.transpose(2, 1, 0)
    cb = conv_b.reshape(1, F)
    LOG2E = 1.4426950408889634
    u4T = (U4_w.T * LOG2E).astype(jnp.bfloat16)
    fw = jnp.concatenate([final4t_w.T, final4_w[:, :F].T,
                          final4t_b.reshape(1, Y)], axis=0)
    wB = final4_w[:, F:]
    gb1 = gcn_b.reshape(1, F)
    sel = (jax.lax.broadcasted_iota(jnp.int32, (B * F, B), 0) // F
           == jax.lax.broadcasted_iota(jnp.int32, (B * F, B), 1)
           ).astype(jnp.float32)
    f4b = final4_b.reshape(1, Y)

    from functools import partial
    U = 25
    hp, hpT = pl.pallas_call(
        partial(_conv_body, L, LP, E, F, K, U),
        grid_spec=pltpu.PrefetchScalarGridSpec(
            num_scalar_prefetch=1,
            grid=(B,),
            in_specs=[
                pl.BlockSpec(memory_space=pl.ANY),
                pl.BlockSpec((K, E, F), lambda b, xf: (0, 0, 0)),
                pl.BlockSpec((1, F), lambda b, xf: (0, 0)),
            ],
            out_specs=[
                pl.BlockSpec((1, LP, F), lambda b, xf: (b, 0, 0)),
                pl.BlockSpec((1, F + 1, LP), lambda b, xf: (b, 0, 0)),
            ],
            scratch_shapes=[
                pltpu.VMEM((V, 1, E), jnp.float32),
                pltpu.VMEM((LP + K - 1, 1, E), jnp.float32),
                pltpu.SemaphoreType.DMA,
            ],
        ),
        out_shape=[
            jax.ShapeDtypeStruct((B, LP, F), jnp.bfloat16),
            jax.ShapeDtypeStruct((B, F + 1, LP), jnp.bfloat16),
        ],
        compiler_params=pltpu.CompilerParams(
            dimension_semantics=("arbitrary",),
            vmem_limit_bytes=48 * 1024 * 1024),
        name="conv_tanh",
    )(xf, tbl3, wt, cb)

    S_flat, y4t3, y4a3 = pl.pallas_call(
        partial(_attn_body, B, L, F, YB, 1),
        grid=(NY, B),
        in_specs=[
            pl.BlockSpec((B, LP, F), lambda i, b: (0, 0, 0)),
            pl.BlockSpec((B, F + 1, LP), lambda i, b: (0, 0, 0)),
            pl.BlockSpec((F, YB), lambda i, b: (0, i)),
            pl.BlockSpec((2 * F + 1, YB), lambda i, b: (0, i)),
            pl.BlockSpec((F, F), lambda i, b: (0, 0)),
        ],
        out_specs=[
            pl.BlockSpec((YB, B * F), lambda i, b: (i, 0)),
            pl.BlockSpec((1, 1, YB), lambda i, b: (b, 0, i)),
            pl.BlockSpec((1, 1, YB), lambda i, b: (b, 0, i)),
        ],
        out_shape=[
            jax.ShapeDtypeStruct((Y, B * F), jnp.bfloat16),
            jax.ShapeDtypeStruct((B, 1, Y), jnp.float32),
            jax.ShapeDtypeStruct((B, 1, Y), jnp.float32),
        ],
        compiler_params=pltpu.CompilerParams(
            dimension_semantics=("parallel", "arbitrary"),
            vmem_limit_bytes=48 * 1024 * 1024),
        name="attn_pool",
    )(hp, hpT, u4T, fw, gcn_w)

    y4t = y4t3.reshape(B, Y)
    y4a = y4a3.reshape(B, Y)

    y4 = pl.pallas_call(
        partial(_gcn_body, B, F),
        grid=(NI,),
        in_specs=[
            pl.BlockSpec((IB, Y), lambda i: (i, 0)),
            pl.BlockSpec((Y, B * F), lambda i: (0, 0)),
            pl.BlockSpec((IB, F), lambda i: (i, 0)),
            pl.BlockSpec((1, F), lambda i: (0, 0)),
            pl.BlockSpec((B * F, B), lambda i: (0, 0)),
            pl.BlockSpec((B, IB), lambda i: (0, i)),
            pl.BlockSpec((1, IB), lambda i: (0, i)),
        ],
        out_specs=pl.BlockSpec((B, IB), lambda i: (0, i)),
        out_shape=jax.ShapeDtypeStruct((B, Y), jnp.float32),
        compiler_params=pltpu.CompilerParams(
            dimension_semantics=("parallel",),
            vmem_limit_bytes=56 * 1024 * 1024),
        name="gcn_score",
    )(adj, S_flat, wB, gb1, sel, y4a, f4b)

    return y4t, y4


# speedup vs baseline: 1.2920x; 1.0507x over previous
import jax
import jax.numpy as jnp
from jax.experimental import pallas as pl
from jax.experimental.pallas import tpu as pltpu


def _conv_body(L, LP, E, F, K, U, xf_ref, tbl_ref, wt_ref, b_ref,
               hp_ref, hpT_ref, tbl_v, emb_s, sem):
    b = pl.program_id(0)
    half = K // 2

    @pl.when(b == 0)
    def _():
        cp = pltpu.make_async_copy(tbl_ref, tbl_v, sem)
        cp.start()
        cp.wait()

    emb_s[0:half, 0, :] = jnp.zeros((half, E), jnp.float32)
    nz = emb_s.shape[0] - half - L
    emb_s[half + L:, 0, :] = jnp.zeros((nz, E), jnp.float32)

    base = b * L

    def gather_chunk(o, carry):
        s = o * U
        for u in range(U):
            idx = xf_ref[base + s + u]
            emb_s[pl.ds(half + s + u, 1)] = tbl_v[pl.ds(idx, 1)]
        return carry

    jax.lax.fori_loop(0, L // U, gather_chunk, 0)

    e = emb_s[:, 0, :]
    acc = jnp.zeros((LP, F), jnp.float32)
    for k in range(K):
        acc = acc + jnp.dot(e[k:k + LP, :], wt_ref[k],
                            preferred_element_type=jnp.float32)
    h = jnp.tanh(acc + b_ref[...])
    rows = jax.lax.broadcasted_iota(jnp.int32, (LP, F), 0)
    h = jnp.where(rows < L, h, 0.0).astype(jnp.bfloat16)
    hp_ref[0] = h
    ones = jnp.ones((1, h.shape[0]), jnp.bfloat16)
    hpT_ref[0] = jnp.concatenate([h.T, ones], axis=0)


def _attn_body(B, L, F, PB,
               hp_ref, hpT_ref, u4T_ref, fw_ref, gcn_w_ref,
               S_ref, y4t_ref, y4a_ref):
    p = pl.program_id(1)
    stripes = []
    for bb in range(PB):
        b = p * PB + bb
        hp = hp_ref[b][:L]
        hpT1 = hpT_ref[b][:, :L]
        sT = jnp.dot(hp, u4T_ref[...],
                     preferred_element_type=jnp.float32)
        e = jnp.exp2(sT).astype(jnp.bfloat16)
        m1 = jnp.dot(hpT1, e,
                     preferred_element_type=jnp.float32)
        m4tT = m1[:F] * (1.0 / m1[F:F + 1])
        y4t_ref[bb, :, :] = (jnp.sum(m4tT * fw_ref[0:F], axis=0,
                                     keepdims=True)
                             + fw_ref[2 * F:2 * F + 1])
        y4a_ref[bb, :, :] = jnp.sum(m4tT * fw_ref[F:2 * F], axis=0,
                                    keepdims=True)
        sup = jax.lax.dot_general(
            m4tT, gcn_w_ref[...], (((0,), (0,)), ((), ())),
            preferred_element_type=jnp.float32)
        stripes.append(sup.astype(jnp.bfloat16))
    for j in range(B // PB):
        @pl.when(p == j)
        def _():
            for bb in range(PB):
                c = (j * PB + bb) * F
                S_ref[:, c:c + F] = stripes[bb]


def _gcn_body(B, F, adj_ref, S_ref, wB_ref, gb_ref, sel_ref, y4a_ref,
              f4b_ref, y4_ref):
    out1 = jax.lax.dot_general(
        adj_ref[...], S_ref[...], (((1,), (0,)), ((), ())),
        preferred_element_type=jnp.float32)
    out1 = out1 + jnp.tile(gb_ref[...], (1, B))
    out1 = jnp.where(out1 >= 0.0, out1, 0.2 * out1)
    prod = out1 * jnp.tile(wB_ref[...], (1, B))
    cols = jnp.dot(prod, sel_ref[...],
                   preferred_element_type=jnp.float32)
    y4_ref[...] = y4a_ref[...] + cols.T + f4b_ref[...]


def kernel(x, target, embed_w, conv_w, conv_b, U4_w, gcn_w, gcn_b, adj,
           final4t_w, final4t_b, final4_w, final4_b):
    B, L = x.shape
    V, E = embed_w.shape
    F = conv_w.shape[0]
    K = conv_w.shape[2]
    Y = U4_w.shape[0]
    LP = ((L + 127) // 128) * 128
    YB = 512
    NY = (Y + YB - 1) // YB
    IB = 256
    NI = (Y + IB - 1) // IB
    half = K // 2

    xf = x.astype(jnp.int32).reshape(-1)
    tbl3 = embed_w.reshape(V, 1, E)
    wt = conv_w.transpose(2, 1, 0)
    cb = conv_b.reshape(1, F)
    LOG2E = 1.4426950408889634
    u4T = (U4_w.T * LOG2E).astype(jnp.bfloat16)
    fw = jnp.concatenate([final4t_w.T, final4_w[:, :F].T,
                          final4t_b.reshape(1, Y)], axis=0)
    wB = final4_w[:, F:]
    gb1 = gcn_b.reshape(1, F)
    sel = (jax.lax.broadcasted_iota(jnp.int32, (B * F, B), 0) // F
           == jax.lax.broadcasted_iota(jnp.int32, (B * F, B), 1)
           ).astype(jnp.float32)
    f4b = final4_b.reshape(1, Y)

    from functools import partial
    U = 25
    hp, hpT = pl.pallas_call(
        partial(_conv_body, L, LP, E, F, K, U),
        grid_spec=pltpu.PrefetchScalarGridSpec(
            num_scalar_prefetch=1,
            grid=(B,),
            in_specs=[
                pl.BlockSpec(memory_space=pl.ANY),
                pl.BlockSpec((K, E, F), lambda b, xf: (0, 0, 0)),
                pl.BlockSpec((1, F), lambda b, xf: (0, 0)),
            ],
            out_specs=[
                pl.BlockSpec((1, LP, F), lambda b, xf: (b, 0, 0)),
                pl.BlockSpec((1, F + 1, LP), lambda b, xf: (b, 0, 0)),
            ],
            scratch_shapes=[
                pltpu.VMEM((V, 1, E), jnp.float32),
                pltpu.VMEM((LP + K - 1, 1, E), jnp.float32),
                pltpu.SemaphoreType.DMA,
            ],
        ),
        out_shape=[
            jax.ShapeDtypeStruct((B, LP, F), jnp.bfloat16),
            jax.ShapeDtypeStruct((B, F + 1, LP), jnp.bfloat16),
        ],
        compiler_params=pltpu.CompilerParams(
            dimension_semantics=("arbitrary",),
            vmem_limit_bytes=48 * 1024 * 1024),
        name="conv_tanh",
    )(xf, tbl3, wt, cb)

    PB = 2
    S_flat, y4t3, y4a3 = pl.pallas_call(
        partial(_attn_body, B, L, F, PB),
        grid=(NY, B // PB),
        in_specs=[
            pl.BlockSpec((B, LP, F), lambda i, b: (0, 0, 0)),
            pl.BlockSpec((B, F + 1, LP), lambda i, b: (0, 0, 0)),
            pl.BlockSpec((F, YB), lambda i, b: (0, i)),
            pl.BlockSpec((2 * F + 1, YB), lambda i, b: (0, i)),
            pl.BlockSpec((F, F), lambda i, b: (0, 0)),
        ],
        out_specs=[
            pl.BlockSpec((YB, B * F), lambda i, b: (i, 0)),
            pl.BlockSpec((PB, 1, YB), lambda i, b: (b, 0, i)),
            pl.BlockSpec((PB, 1, YB), lambda i, b: (b, 0, i)),
        ],
        out_shape=[
            jax.ShapeDtypeStruct((Y, B * F), jnp.bfloat16),
            jax.ShapeDtypeStruct((B, 1, Y), jnp.float32),
            jax.ShapeDtypeStruct((B, 1, Y), jnp.float32),
        ],
        compiler_params=pltpu.CompilerParams(
            dimension_semantics=("parallel", "arbitrary"),
            vmem_limit_bytes=48 * 1024 * 1024),
        name="attn_pool",
    )(hp, hpT, u4T, fw, gcn_w)

    y4t = y4t3.reshape(B, Y)
    y4a = y4a3.reshape(B, Y)

    y4 = pl.pallas_call(
        partial(_gcn_body, B, F),
        grid=(NI,),
        in_specs=[
            pl.BlockSpec((IB, Y), lambda i: (i, 0)),
            pl.BlockSpec((Y, B * F), lambda i: (0, 0)),
            pl.BlockSpec((IB, F), lambda i: (i, 0)),
            pl.BlockSpec((1, F), lambda i: (0, 0)),
            pl.BlockSpec((B * F, B), lambda i: (0, 0)),
            pl.BlockSpec((B, IB), lambda i: (0, i)),
            pl.BlockSpec((1, IB), lambda i: (0, i)),
        ],
        out_specs=pl.BlockSpec((B, IB), lambda i: (0, i)),
        out_shape=jax.ShapeDtypeStruct((B, Y), jnp.float32),
        compiler_params=pltpu.CompilerParams(
            dimension_semantics=("parallel",),
            vmem_limit_bytes=56 * 1024 * 1024),
        name="gcn_score",
    )(adj, S_flat, wB, gb1, sel, y4a, f4b)

    return y4t, y4


# speedup vs baseline: 1.3246x; 1.0252x over previous
import jax
import jax.numpy as jnp
from jax.experimental import pallas as pl
from jax.experimental.pallas import tpu as pltpu


def _conv_body(L, LP, E, F, K, U, xf_ref, tbl_ref, wt_ref, b_ref,
               hp_ref, hpT_ref, tbl_v, emb_s, sem):
    b = pl.program_id(0)
    half = K // 2

    @pl.when(b == 0)
    def _():
        cp = pltpu.make_async_copy(tbl_ref, tbl_v, sem)
        cp.start()
        cp.wait()

    emb_s[0:half, 0, :] = jnp.zeros((half, E), jnp.float32)
    nz = emb_s.shape[0] - half - L
    emb_s[half + L:, 0, :] = jnp.zeros((nz, E), jnp.float32)

    base = b * L

    def gather_chunk(o, carry):
        s = o * U
        for u in range(U):
            idx = xf_ref[base + s + u]
            emb_s[pl.ds(half + s + u, 1)] = tbl_v[pl.ds(idx, 1)]
        return carry

    jax.lax.fori_loop(0, L // U, gather_chunk, 0)

    e = emb_s[:, 0, :]
    acc = jnp.zeros((LP, F), jnp.float32)
    for k in range(K):
        acc = acc + jnp.dot(e[k:k + LP, :], wt_ref[k],
                            preferred_element_type=jnp.float32)
    h = jnp.tanh(acc + b_ref[...])
    rows = jax.lax.broadcasted_iota(jnp.int32, (LP, F), 0)
    h = jnp.where(rows < L, h, 0.0).astype(jnp.bfloat16)
    hp_ref[0] = h
    ones = jnp.ones((1, h.shape[0]), jnp.bfloat16)
    hpT_ref[0] = jnp.concatenate([h.T, ones], axis=0)


def _attn_body(B, L, F, PB,
               hp_ref, hpT_ref, u4T_ref, fw_ref, gcn_w_ref,
               S_ref, y4t_ref, y4a_ref):
    p = pl.program_id(1)
    stripes = []
    for bb in range(PB):
        b = p * PB + bb
        hp = hp_ref[b][:L]
        hpT1 = hpT_ref[b][:, :L]
        sT = jnp.dot(hp, u4T_ref[...],
                     preferred_element_type=jnp.float32)
        e = jnp.exp2(sT).astype(jnp.bfloat16)
        m1 = jnp.dot(hpT1, e,
                     preferred_element_type=jnp.float32)
        m4tT = m1[:F] * (1.0 / m1[F:F + 1])
        y4t_ref[bb, :, :] = (jnp.sum(m4tT * fw_ref[0:F], axis=0,
                                     keepdims=True)
                             + fw_ref[2 * F:2 * F + 1])
        y4a_ref[bb, :, :] = jnp.sum(m4tT * fw_ref[F:2 * F], axis=0,
                                    keepdims=True)
        sup = jax.lax.dot_general(
            m4tT, gcn_w_ref[...], (((0,), (0,)), ((), ())),
            preferred_element_type=jnp.float32)
        stripes.append(sup.astype(jnp.bfloat16))
    for j in range(B // PB):
        @pl.when(p == j)
        def _():
            for bb in range(PB):
                c = (j * PB + bb) * F
                S_ref[:, c:c + F] = stripes[bb]


def _gcn_body(B, F, adj_ref, S_ref, wB_ref, gb_ref, sel_ref, y4a_ref,
              f4b_ref, y4_ref):
    out1 = jax.lax.dot_general(
        adj_ref[...], S_ref[...], (((1,), (0,)), ((), ())),
        preferred_element_type=jnp.float32)
    out1 = out1 + jnp.tile(gb_ref[...], (1, B))
    out1 = jnp.where(out1 >= 0.0, out1, 0.2 * out1)
    prod = out1 * jnp.tile(wB_ref[...], (1, B))
    cols = jnp.dot(prod, sel_ref[...],
                   preferred_element_type=jnp.float32)
    y4_ref[...] = y4a_ref[...] + cols.T + f4b_ref[...]


def kernel(x, target, embed_w, conv_w, conv_b, U4_w, gcn_w, gcn_b, adj,
           final4t_w, final4t_b, final4_w, final4_b):
    B, L = x.shape
    V, E = embed_w.shape
    F = conv_w.shape[0]
    K = conv_w.shape[2]
    Y = U4_w.shape[0]
    LP = ((L + 127) // 128) * 128
    YB = 512
    NY = (Y + YB - 1) // YB
    IB = 256
    NI = (Y + IB - 1) // IB
    half = K // 2

    xf = x.astype(jnp.int32).reshape(-1)
    tbl3 = embed_w.reshape(V, 1, E)
    wt = conv_w.transpose(2, 1, 0)
    cb = conv_b.reshape(1, F)
    LOG2E = 1.4426950408889634
    u4T = (U4_w.T * LOG2E).astype(jnp.bfloat16)
    fw = jnp.concatenate([final4t_w.T, final4_w[:, :F].T,
                          final4t_b.reshape(1, Y)], axis=0)
    wB = final4_w[:, F:]
    gb1 = gcn_b.reshape(1, F)
    sel = (jax.lax.broadcasted_iota(jnp.int32, (B * F, B), 0) // F
           == jax.lax.broadcasted_iota(jnp.int32, (B * F, B), 1)
           ).astype(jnp.float32)
    f4b = final4_b.reshape(1, Y)

    from functools import partial
    U = 25
    hp, hpT = pl.pallas_call(
        partial(_conv_body, L, LP, E, F, K, U),
        grid_spec=pltpu.PrefetchScalarGridSpec(
            num_scalar_prefetch=1,
            grid=(B,),
            in_specs=[
                pl.BlockSpec(memory_space=pl.ANY),
                pl.BlockSpec((K, E, F), lambda b, xf: (0, 0, 0)),
                pl.BlockSpec((1, F), lambda b, xf: (0, 0)),
            ],
            out_specs=[
                pl.BlockSpec((1, LP, F), lambda b, xf: (b, 0, 0)),
                pl.BlockSpec((1, F + 1, LP), lambda b, xf: (b, 0, 0)),
            ],
            scratch_shapes=[
                pltpu.VMEM((V, 1, E), jnp.float32),
                pltpu.VMEM((LP + K - 1, 1, E), jnp.float32),
                pltpu.SemaphoreType.DMA,
            ],
        ),
        out_shape=[
            jax.ShapeDtypeStruct((B, LP, F), jnp.bfloat16),
            jax.ShapeDtypeStruct((B, F + 1, LP), jnp.bfloat16),
        ],
        compiler_params=pltpu.CompilerParams(
            dimension_semantics=("arbitrary",),
            vmem_limit_bytes=48 * 1024 * 1024),
        name="conv_tanh",
    )(xf, tbl3, wt, cb)

    PB = 4
    S_flat, y4t3, y4a3 = pl.pallas_call(
        partial(_attn_body, B, L, F, PB),
        grid=(NY, B // PB),
        in_specs=[
            pl.BlockSpec((B, LP, F), lambda i, b: (0, 0, 0)),
            pl.BlockSpec((B, F + 1, LP), lambda i, b: (0, 0, 0)),
            pl.BlockSpec((F, YB), lambda i, b: (0, i)),
            pl.BlockSpec((2 * F + 1, YB), lambda i, b: (0, i)),
            pl.BlockSpec((F, F), lambda i, b: (0, 0)),
        ],
        out_specs=[
            pl.BlockSpec((YB, B * F), lambda i, b: (i, 0)),
            pl.BlockSpec((PB, 1, YB), lambda i, b: (b, 0, i)),
            pl.BlockSpec((PB, 1, YB), lambda i, b: (b, 0, i)),
        ],
        out_shape=[
            jax.ShapeDtypeStruct((Y, B * F), jnp.bfloat16),
            jax.ShapeDtypeStruct((B, 1, Y), jnp.float32),
            jax.ShapeDtypeStruct((B, 1, Y), jnp.float32),
        ],
        compiler_params=pltpu.CompilerParams(
            dimension_semantics=("parallel", "arbitrary"),
            vmem_limit_bytes=48 * 1024 * 1024),
        name="attn_pool",
    )(hp, hpT, u4T, fw, gcn_w)

    y4t = y4t3.reshape(B, Y)
    y4a = y4a3.reshape(B, Y)

    y4 = pl.pallas_call(
        partial(_gcn_body, B, F),
        grid=(NI,),
        in_specs=[
            pl.BlockSpec((IB, Y), lambda i: (i, 0)),
            pl.BlockSpec((Y, B * F), lambda i: (0, 0)),
            pl.BlockSpec((IB, F), lambda i: (i, 0)),
            pl.BlockSpec((1, F), lambda i: (0, 0)),
            pl.BlockSpec((B * F, B), lambda i: (0, 0)),
            pl.BlockSpec((B, IB), lambda i: (0, i)),
            pl.BlockSpec((1, IB), lambda i: (0, i)),
        ],
        out_specs=pl.BlockSpec((B, IB), lambda i: (0, i)),
        out_shape=jax.ShapeDtypeStruct((B, Y), jnp.float32),
        compiler_params=pltpu.CompilerParams(
            dimension_semantics=("parallel",),
            vmem_limit_bytes=56 * 1024 * 1024),
        name="gcn_score",
    )(adj, S_flat, wB, gb1, sel, y4a, f4b)

    return y4t, y4


# speedup vs baseline: 1.3616x; 1.0279x over previous
import jax
import jax.numpy as jnp
from jax.experimental import pallas as pl
from jax.experimental.pallas import tpu as pltpu


def _conv_body(L, LP, E, F, K, U, xf_ref, tbl_ref, wt_ref, b_ref,
               hp_ref, hpT_ref, tbl_v, emb_s, sem):
    b = pl.program_id(0)
    half = K // 2

    @pl.when(b == 0)
    def _():
        cp = pltpu.make_async_copy(tbl_ref, tbl_v, sem)
        cp.start()
        cp.wait()

    emb_s[0:half, 0, :] = jnp.zeros((half, E), jnp.float32)
    nz = emb_s.shape[0] - half - L
    emb_s[half + L:, 0, :] = jnp.zeros((nz, E), jnp.float32)

    base = b * L

    def gather_chunk(o, carry):
        s = o * U
        for u in range(U):
            idx = xf_ref[base + s + u]
            emb_s[pl.ds(half + s + u, 1)] = tbl_v[pl.ds(idx, 1)]
        return carry

    jax.lax.fori_loop(0, L // U, gather_chunk, 0)

    e = emb_s[:, 0, :]
    acc = jnp.zeros((LP, F), jnp.float32)
    for k in range(K):
        acc = acc + jnp.dot(e[k:k + LP, :], wt_ref[k],
                            preferred_element_type=jnp.float32)
    h = jnp.tanh(acc + b_ref[...])
    rows = jax.lax.broadcasted_iota(jnp.int32, (LP, F), 0)
    h = jnp.where(rows < L, h, 0.0).astype(jnp.bfloat16)
    hp_ref[0] = h
    ones = jnp.ones((1, h.shape[0]), jnp.bfloat16)
    hpT_ref[0] = jnp.concatenate([h.T, ones], axis=0)


def _attn_body(B, L, F, PB,
               hp_ref, hpT_ref, u4T_ref, fw_ref, gcn_w_ref,
               S_ref, y4t_ref, y4a_ref):
    p = pl.program_id(1)
    stripes = []
    for bb in range(PB):
        b = p * PB + bb
        hp = hp_ref[b][:L]
        hpT1 = hpT_ref[b][:, :L]
        sT = jnp.dot(hp, u4T_ref[...],
                     preferred_element_type=jnp.float32)
        e = jnp.exp2(sT).astype(jnp.bfloat16)
        m1 = jnp.dot(hpT1, e,
                     preferred_element_type=jnp.float32)
        m4tT = m1[:F] * (1.0 / m1[F:F + 1])
        y4t_ref[bb, :, :] = (jnp.sum(m4tT * fw_ref[0:F], axis=0,
                                     keepdims=True)
                             + fw_ref[2 * F:2 * F + 1])
        y4a_ref[bb, :, :] = jnp.sum(m4tT * fw_ref[F:2 * F], axis=0,
                                    keepdims=True)
        sup = jax.lax.dot_general(
            m4tT, gcn_w_ref[...], (((0,), (0,)), ((), ())),
            preferred_element_type=jnp.float32)
        stripes.append(sup.astype(jnp.bfloat16))
    for j in range(B // PB):
        @pl.when(p == j)
        def _():
            for bb in range(PB):
                c = (j * PB + bb) * F
                S_ref[:, c:c + F] = stripes[bb]


def _gcn_body(B, F, adj_ref, S_ref, wB_ref, gb_ref, sel_ref, y4a_ref,
              f4b_ref, y4_ref):
    out1 = jax.lax.dot_general(
        adj_ref[...], S_ref[...], (((1,), (0,)), ((), ())),
        preferred_element_type=jnp.float32)
    out1 = out1 + jnp.tile(gb_ref[...], (1, B))
    out1 = jnp.where(out1 >= 0.0, out1, 0.2 * out1)
    prod = out1 * jnp.tile(wB_ref[...], (1, B))
    cols = jnp.dot(prod, sel_ref[...],
                   preferred_element_type=jnp.float32)
    y4_ref[...] = y4a_ref[...] + cols.T + f4b_ref[...]


def kernel(x, target, embed_w, conv_w, conv_b, U4_w, gcn_w, gcn_b, adj,
           final4t_w, final4t_b, final4_w, final4_b):
    B, L = x.shape
    V, E = embed_w.shape
    F = conv_w.shape[0]
    K = conv_w.shape[2]
    Y = U4_w.shape[0]
    LP = ((L + 127) // 128) * 128
    YB = 512
    NY = (Y + YB - 1) // YB
    IB = 256
    NI = (Y + IB - 1) // IB
    half = K // 2

    xf = x.astype(jnp.int32).reshape(-1)
    tbl3 = embed_w.reshape(V, 1, E)
    wt = conv_w.transpose(2, 1, 0)
    cb = conv_b.reshape(1, F)
    LOG2E = 1.4426950408889634
    u4T = (U4_w.T * LOG2E).astype(jnp.bfloat16)
    fw = jnp.concatenate([final4t_w.T, final4_w[:, :F].T,
                          final4t_b.reshape(1, Y)], axis=0)
    wB = final4_w[:, F:]
    gb1 = gcn_b.reshape(1, F)
    sel = (jax.lax.broadcasted_iota(jnp.int32, (B * F, B), 0) // F
           == jax.lax.broadcasted_iota(jnp.int32, (B * F, B), 1)
           ).astype(jnp.float32)
    f4b = final4_b.reshape(1, Y)

    from functools import partial
    U = 25
    hp, hpT = pl.pallas_call(
        partial(_conv_body, L, LP, E, F, K, U),
        grid_spec=pltpu.PrefetchScalarGridSpec(
            num_scalar_prefetch=1,
            grid=(B,),
            in_specs=[
                pl.BlockSpec(memory_space=pl.ANY),
                pl.BlockSpec((K, E, F), lambda b, xf: (0, 0, 0)),
                pl.BlockSpec((1, F), lambda b, xf: (0, 0)),
            ],
            out_specs=[
                pl.BlockSpec((1, LP, F), lambda b, xf: (b, 0, 0)),
                pl.BlockSpec((1, F + 1, LP), lambda b, xf: (b, 0, 0)),
            ],
            scratch_shapes=[
                pltpu.VMEM((V, 1, E), jnp.float32),
                pltpu.VMEM((LP + K - 1, 1, E), jnp.float32),
                pltpu.SemaphoreType.DMA,
            ],
        ),
        out_shape=[
            jax.ShapeDtypeStruct((B, LP, F), jnp.bfloat16),
            jax.ShapeDtypeStruct((B, F + 1, LP), jnp.bfloat16),
        ],
        compiler_params=pltpu.CompilerParams(
            dimension_semantics=("arbitrary",),
            vmem_limit_bytes=48 * 1024 * 1024),
        name="conv_tanh",
    )(xf, tbl3, wt, cb)

    PB = 8
    S_flat, y4t3, y4a3 = pl.pallas_call(
        partial(_attn_body, B, L, F, PB),
        grid=(NY, B // PB),
        in_specs=[
            pl.BlockSpec((B, LP, F), lambda i, b: (0, 0, 0)),
            pl.BlockSpec((B, F + 1, LP), lambda i, b: (0, 0, 0)),
            pl.BlockSpec((F, YB), lambda i, b: (0, i)),
            pl.BlockSpec((2 * F + 1, YB), lambda i, b: (0, i)),
            pl.BlockSpec((F, F), lambda i, b: (0, 0)),
        ],
        out_specs=[
            pl.BlockSpec((YB, B * F), lambda i, b: (i, 0)),
            pl.BlockSpec((PB, 1, YB), lambda i, b: (b, 0, i)),
            pl.BlockSpec((PB, 1, YB), lambda i, b: (b, 0, i)),
        ],
        out_shape=[
            jax.ShapeDtypeStruct((Y, B * F), jnp.bfloat16),
            jax.ShapeDtypeStruct((B, 1, Y), jnp.float32),
            jax.ShapeDtypeStruct((B, 1, Y), jnp.float32),
        ],
        compiler_params=pltpu.CompilerParams(
            dimension_semantics=("parallel", "arbitrary"),
            vmem_limit_bytes=48 * 1024 * 1024),
        name="attn_pool",
    )(hp, hpT, u4T, fw, gcn_w)

    y4t = y4t3.reshape(B, Y)
    y4a = y4a3.reshape(B, Y)

    y4 = pl.pallas_call(
        partial(_gcn_body, B, F),
        grid=(NI,),
        in_specs=[
            pl.BlockSpec((IB, Y), lambda i: (i, 0)),
            pl.BlockSpec((Y, B * F), lambda i: (0, 0)),
            pl.BlockSpec((IB, F), lambda i: (i, 0)),
            pl.BlockSpec((1, F), lambda i: (0, 0)),
            pl.BlockSpec((B * F, B), lambda i: (0, 0)),
            pl.BlockSpec((B, IB), lambda i: (0, i)),
            pl.BlockSpec((1, IB), lambda i: (0, i)),
        ],
        out_specs=pl.BlockSpec((B, IB), lambda i: (0, i)),
        out_shape=jax.ShapeDtypeStruct((B, Y), jnp.float32),
        compiler_params=pltpu.CompilerParams(
            dimension_semantics=("parallel",),
            vmem_limit_bytes=56 * 1024 * 1024),
        name="gcn_score",
    )(adj, S_flat, wB, gb1, sel, y4a, f4b)

    return y4t, y4


# speedup vs baseline: 1.3810x; 1.0142x over previous
import jax
import jax.numpy as jnp
from jax.experimental import pallas as pl
from jax.experimental.pallas import tpu as pltpu


def _conv_body(L, LP, E, F, K, U, xf_ref, tbl_ref, wt_ref, b_ref,
               hp_ref, hpT_ref, tbl_v, emb_s, sem):
    b = pl.program_id(0)
    half = K // 2

    @pl.when(b == 0)
    def _():
        cp = pltpu.make_async_copy(tbl_ref, tbl_v, sem)
        cp.start()
        cp.wait()

    emb_s[0:half, 0, :] = jnp.zeros((half, E), jnp.float32)
    nz = emb_s.shape[0] - half - L
    emb_s[half + L:, 0, :] = jnp.zeros((nz, E), jnp.float32)

    base = b * L

    def gather_chunk(o, carry):
        s = o * U
        for u in range(U):
            idx = xf_ref[base + s + u]
            emb_s[pl.ds(half + s + u, 1)] = tbl_v[pl.ds(idx, 1)]
        return carry

    jax.lax.fori_loop(0, L // U, gather_chunk, 0)

    e = emb_s[:, 0, :]
    acc = jnp.zeros((LP, F), jnp.float32)
    for k in range(K):
        acc = acc + jnp.dot(e[k:k + LP, :], wt_ref[k],
                            preferred_element_type=jnp.float32)
    h = jnp.tanh(acc + b_ref[...])
    rows = jax.lax.broadcasted_iota(jnp.int32, (LP, F), 0)
    h = jnp.where(rows < L, h, 0.0).astype(jnp.bfloat16)
    hp_ref[0] = h
    ones = jnp.ones((1, h.shape[0]), jnp.bfloat16)
    hpT_ref[0] = jnp.concatenate([h.T, ones], axis=0)


def _attn_body(B, L, F, PB,
               hp_ref, hpT_ref, u4T_ref, fw_ref, gcn_w_ref,
               S_ref, y4t_ref, y4a_ref):
    p = pl.program_id(1)
    stripes = []
    for bb in range(PB):
        b = p * PB + bb
        hp = hp_ref[b][:L]
        hpT1 = hpT_ref[b][:, :L]
        sT = jnp.dot(hp, u4T_ref[...],
                     preferred_element_type=jnp.float32)
        e = jnp.exp2(sT).astype(jnp.bfloat16)
        m1 = jnp.dot(hpT1, e,
                     preferred_element_type=jnp.float32)
        m4tT = m1[:F] * (1.0 / m1[F:F + 1])
        y4t_ref[bb, :, :] = (jnp.sum(m4tT * fw_ref[0:F], axis=0,
                                     keepdims=True)
                             + fw_ref[2 * F:2 * F + 1])
        y4a_ref[bb, :, :] = jnp.sum(m4tT * fw_ref[F:2 * F], axis=0,
                                    keepdims=True)
        sup = jax.lax.dot_general(
            m4tT, gcn_w_ref[...], (((0,), (0,)), ((), ())),
            preferred_element_type=jnp.float32)
        stripes.append(sup.astype(jnp.bfloat16))
    for j in range(B // PB):
        @pl.when(p == j)
        def _():
            for bb in range(PB):
                c = (j * PB + bb) * F
                S_ref[:, c:c + F] = stripes[bb]


def _gcn_body(B, F, adj_ref, S_ref, wB_ref, gb_ref, sel_ref, y4a_ref,
              f4b_ref, y4_ref):
    out1 = jax.lax.dot_general(
        adj_ref[...], S_ref[...], (((1,), (0,)), ((), ())),
        preferred_element_type=jnp.float32)
    out1 = out1 + jnp.tile(gb_ref[...], (1, B))
    out1 = jnp.where(out1 >= 0.0, out1, 0.2 * out1)
    prod = out1 * jnp.tile(wB_ref[...], (1, B))
    cols = jnp.dot(prod, sel_ref[...],
                   preferred_element_type=jnp.float32)
    y4_ref[...] = y4a_ref[...] + cols.T + f4b_ref[...]


def kernel(x, target, embed_w, conv_w, conv_b, U4_w, gcn_w, gcn_b, adj,
           final4t_w, final4t_b, final4_w, final4_b):
    B, L = x.shape
    V, E = embed_w.shape
    F = conv_w.shape[0]
    K = conv_w.shape[2]
    Y = U4_w.shape[0]
    LP = ((L + 127) // 128) * 128
    YB = 512
    NY = (Y + YB - 1) // YB
    IB = 384
    NI = (Y + IB - 1) // IB
    half = K // 2

    xf = x.astype(jnp.int32).reshape(-1)
    tbl3 = embed_w.reshape(V, 1, E)
    wt = conv_w.transpose(2, 1, 0)
    cb = conv_b.reshape(1, F)
    LOG2E = 1.4426950408889634
    u4T = (U4_w.T * LOG2E).astype(jnp.bfloat16)
    fw = jnp.concatenate([final4t_w.T, final4_w[:, :F].T,
                          final4t_b.reshape(1, Y)], axis=0)
    wB = final4_w[:, F:]
    gb1 = gcn_b.reshape(1, F)
    sel = (jax.lax.broadcasted_iota(jnp.int32, (B * F, B), 0) // F
           == jax.lax.broadcasted_iota(jnp.int32, (B * F, B), 1)
           ).astype(jnp.float32)
    f4b = final4_b.reshape(1, Y)

    from functools import partial
    U = 50
    hp, hpT = pl.pallas_call(
        partial(_conv_body, L, LP, E, F, K, U),
        grid_spec=pltpu.PrefetchScalarGridSpec(
            num_scalar_prefetch=1,
            grid=(B,),
            in_specs=[
                pl.BlockSpec(memory_space=pl.ANY),
                pl.BlockSpec((K, E, F), lambda b, xf: (0, 0, 0)),
                pl.BlockSpec((1, F), lambda b, xf: (0, 0)),
            ],
            out_specs=[
                pl.BlockSpec((1, LP, F), lambda b, xf: (b, 0, 0)),
                pl.BlockSpec((1, F + 1, LP), lambda b, xf: (b, 0, 0)),
            ],
            scratch_shapes=[
                pltpu.VMEM((V, 1, E), jnp.float32),
                pltpu.VMEM((LP + K - 1, 1, E), jnp.float32),
                pltpu.SemaphoreType.DMA,
            ],
        ),
        out_shape=[
            jax.ShapeDtypeStruct((B, LP, F), jnp.bfloat16),
            jax.ShapeDtypeStruct((B, F + 1, LP), jnp.bfloat16),
        ],
        compiler_params=pltpu.CompilerParams(
            dimension_semantics=("arbitrary",),
            vmem_limit_bytes=48 * 1024 * 1024),
        name="conv_tanh",
    )(xf, tbl3, wt, cb)

    PB = 8
    S_flat, y4t3, y4a3 = pl.pallas_call(
        partial(_attn_body, B, L, F, PB),
        grid=(NY, B // PB),
        in_specs=[
            pl.BlockSpec((B, LP, F), lambda i, b: (0, 0, 0)),
            pl.BlockSpec((B, F + 1, LP), lambda i, b: (0, 0, 0)),
            pl.BlockSpec((F, YB), lambda i, b: (0, i)),
            pl.BlockSpec((2 * F + 1, YB), lambda i, b: (0, i)),
            pl.BlockSpec((F, F), lambda i, b: (0, 0)),
        ],
        out_specs=[
            pl.BlockSpec((YB, B * F), lambda i, b: (i, 0)),
            pl.BlockSpec((PB, 1, YB), lambda i, b: (b, 0, i)),
            pl.BlockSpec((PB, 1, YB), lambda i, b: (b, 0, i)),
        ],
        out_shape=[
            jax.ShapeDtypeStruct((Y, B * F), jnp.bfloat16),
            jax.ShapeDtypeStruct((B, 1, Y), jnp.float32),
            jax.ShapeDtypeStruct((B, 1, Y), jnp.float32),
        ],
        compiler_params=pltpu.CompilerParams(
            dimension_semantics=("parallel", "arbitrary"),
            vmem_limit_bytes=48 * 1024 * 1024),
        name="attn_pool",
    )(hp, hpT, u4T, fw, gcn_w)

    y4t = y4t3.reshape(B, Y)
    y4a = y4a3.reshape(B, Y)

    y4 = pl.pallas_call(
        partial(_gcn_body, B, F),
        grid=(NI,),
        in_specs=[
            pl.BlockSpec((IB, Y), lambda i: (i, 0)),
            pl.BlockSpec((Y, B * F), lambda i: (0, 0)),
            pl.BlockSpec((IB, F), lambda i: (i, 0)),
            pl.BlockSpec((1, F), lambda i: (0, 0)),
            pl.BlockSpec((B * F, B), lambda i: (0, 0)),
            pl.BlockSpec((B, IB), lambda i: (0, i)),
            pl.BlockSpec((1, IB), lambda i: (0, i)),
        ],
        out_specs=pl.BlockSpec((B, IB), lambda i: (0, i)),
        out_shape=jax.ShapeDtypeStruct((B, Y), jnp.float32),
        compiler_params=pltpu.CompilerParams(
            dimension_semantics=("parallel",),
            vmem_limit_bytes=56 * 1024 * 1024),
        name="gcn_score",
    )(adj, S_flat, wB, gb1, sel, y4a, f4b)

    return y4t, y4


# speedup vs baseline: 1.4287x; 1.0345x over previous
import jax
import jax.numpy as jnp
from jax.experimental import pallas as pl
from jax.experimental.pallas import tpu as pltpu


def _conv_body(L, LP, E, F, K, U, xf_ref, tbl_ref, wt_ref, b_ref,
               hp_ref, hpT_ref, tbl_v, emb_s, sem):
    b = pl.program_id(0)
    half = K // 2

    @pl.when(b == 0)
    def _():
        cp = pltpu.make_async_copy(tbl_ref, tbl_v, sem)
        cp.start()
        cp.wait()

    emb_s[0:half, 0, :] = jnp.zeros((half, E), jnp.float32)
    nz = emb_s.shape[0] - half - L
    emb_s[half + L:, 0, :] = jnp.zeros((nz, E), jnp.float32)

    base = b * L

    def gather_chunk(o, carry):
        s = o * U
        for u in range(U):
            idx = xf_ref[base + s + u]
            emb_s[pl.ds(half + s + u, 1)] = tbl_v[pl.ds(idx, 1)]
        return carry

    jax.lax.fori_loop(0, L // U, gather_chunk, 0)

    e = emb_s[:, 0, :]
    acc = jnp.zeros((LP, F), jnp.float32)
    for k in range(K):
        acc = acc + jnp.dot(e[k:k + LP, :], wt_ref[k],
                            preferred_element_type=jnp.float32)
    h = jnp.tanh(acc + b_ref[...])
    rows = jax.lax.broadcasted_iota(jnp.int32, (LP, F), 0)
    h = jnp.where(rows < L, h, 0.0).astype(jnp.bfloat16)
    hp_ref[0] = h
    ones = jnp.ones((1, h.shape[0]), jnp.bfloat16)
    hpT_ref[0] = jnp.concatenate([h.T, ones], axis=0)


def _attn_body(B, L, F, PB,
               hp_ref, hpT_ref, u4T_ref, fw_ref, gcn_w_ref,
               S_ref, y4t_ref, y4a_ref):
    p = pl.program_id(1)
    stripes = []
    for bb in range(PB):
        b = p * PB + bb
        hp = hp_ref[b][:L]
        hpT1 = hpT_ref[b][:, :L]
        sT = jnp.dot(hp, u4T_ref[...],
                     preferred_element_type=jnp.float32)
        e = jnp.exp2(sT).astype(jnp.bfloat16)
        m1 = jnp.dot(hpT1, e,
                     preferred_element_type=jnp.float32)
        m4tT = m1[:F] * (1.0 / m1[F:F + 1])
        y4t_ref[bb, :, :] = (jnp.sum(m4tT * fw_ref[0:F], axis=0,
                                     keepdims=True)
                             + fw_ref[2 * F:2 * F + 1])
        y4a_ref[bb, :, :] = jnp.sum(m4tT * fw_ref[F:2 * F], axis=0,
                                    keepdims=True)
        sup = jax.lax.dot_general(
            m4tT, gcn_w_ref[...], (((0,), (0,)), ((), ())),
            preferred_element_type=jnp.float32)
        stripes.append(sup.astype(jnp.bfloat16))
    for j in range(B // PB):
        @pl.when(p == j)
        def _():
            for bb in range(PB):
                c = (j * PB + bb) * F
                S_ref[:, c:c + F] = stripes[bb]


def _gcn_body(B, F, adj_ref, S_ref, wB_ref, gb_ref, sel_ref, y4a_ref,
              f4b_ref, y4_ref):
    out1 = jax.lax.dot_general(
        adj_ref[...], S_ref[...], (((1,), (0,)), ((), ())),
        preferred_element_type=jnp.float32)
    out1 = out1 + jnp.tile(gb_ref[...], (1, B))
    out1 = jnp.where(out1 >= 0.0, out1, 0.2 * out1)
    prod = out1 * jnp.tile(wB_ref[...], (1, B))
    cols = jnp.dot(prod, sel_ref[...],
                   preferred_element_type=jnp.float32)
    y4_ref[...] = y4a_ref[...] + cols.T + f4b_ref[...]


def kernel(x, target, embed_w, conv_w, conv_b, U4_w, gcn_w, gcn_b, adj,
           final4t_w, final4t_b, final4_w, final4_b):
    B, L = x.shape
    V, E = embed_w.shape
    F = conv_w.shape[0]
    K = conv_w.shape[2]
    Y = U4_w.shape[0]
    LP = ((L + 127) // 128) * 128
    YB = 1024
    NY = (Y + YB - 1) // YB
    IB = 384
    NI = (Y + IB - 1) // IB
    half = K // 2

    xf = x.astype(jnp.int32).reshape(-1)
    tbl3 = embed_w.reshape(V, 1, E)
    wt = conv_w.transpose(2, 1, 0)
    cb = conv_b.reshape(1, F)
    LOG2E = 1.4426950408889634
    u4T = (U4_w.T * LOG2E).astype(jnp.bfloat16)
    fw = jnp.concatenate([final4t_w.T, final4_w[:, :F].T,
                          final4t_b.reshape(1, Y)], axis=0)
    wB = final4_w[:, F:]
    gb1 = gcn_b.reshape(1, F)
    sel = (jax.lax.broadcasted_iota(jnp.int32, (B * F, B), 0) // F
           == jax.lax.broadcasted_iota(jnp.int32, (B * F, B), 1)
           ).astype(jnp.float32)
    f4b = final4_b.reshape(1, Y)

    from functools import partial
    U = 50
    hp, hpT = pl.pallas_call(
        partial(_conv_body, L, LP, E, F, K, U),
        grid_spec=pltpu.PrefetchScalarGridSpec(
            num_scalar_prefetch=1,
            grid=(B,),
            in_specs=[
                pl.BlockSpec(memory_space=pl.ANY),
                pl.BlockSpec((K, E, F), lambda b, xf: (0, 0, 0)),
                pl.BlockSpec((1, F), lambda b, xf: (0, 0)),
            ],
            out_specs=[
                pl.BlockSpec((1, LP, F), lambda b, xf: (b, 0, 0)),
                pl.BlockSpec((1, F + 1, LP), lambda b, xf: (b, 0, 0)),
            ],
            scratch_shapes=[
                pltpu.VMEM((V, 1, E), jnp.float32),
                pltpu.VMEM((LP + K - 1, 1, E), jnp.float32),
                pltpu.SemaphoreType.DMA,
            ],
        ),
        out_shape=[
            jax.ShapeDtypeStruct((B, LP, F), jnp.bfloat16),
            jax.ShapeDtypeStruct((B, F + 1, LP), jnp.bfloat16),
        ],
        compiler_params=pltpu.CompilerParams(
            dimension_semantics=("arbitrary",),
            vmem_limit_bytes=48 * 1024 * 1024),
        name="conv_tanh",
    )(xf, tbl3, wt, cb)

    PB = 8
    S_flat, y4t3, y4a3 = pl.pallas_call(
        partial(_attn_body, B, L, F, PB),
        grid=(NY, B // PB),
        in_specs=[
            pl.BlockSpec((B, LP, F), lambda i, b: (0, 0, 0)),
            pl.BlockSpec((B, F + 1, LP), lambda i, b: (0, 0, 0)),
            pl.BlockSpec((F, YB), lambda i, b: (0, i)),
            pl.BlockSpec((2 * F + 1, YB), lambda i, b: (0, i)),
            pl.BlockSpec((F, F), lambda i, b: (0, 0)),
        ],
        out_specs=[
            pl.BlockSpec((YB, B * F), lambda i, b: (i, 0)),
            pl.BlockSpec((PB, 1, YB), lambda i, b: (b, 0, i)),
            pl.BlockSpec((PB, 1, YB), lambda i, b: (b, 0, i)),
        ],
        out_shape=[
            jax.ShapeDtypeStruct((Y, B * F), jnp.bfloat16),
            jax.ShapeDtypeStruct((B, 1, Y), jnp.float32),
            jax.ShapeDtypeStruct((B, 1, Y), jnp.float32),
        ],
        compiler_params=pltpu.CompilerParams(
            dimension_semantics=("parallel", "arbitrary"),
            vmem_limit_bytes=48 * 1024 * 1024),
        name="attn_pool",
    )(hp, hpT, u4T, fw, gcn_w)

    y4t = y4t3.reshape(B, Y)
    y4a = y4a3.reshape(B, Y)

    y4 = pl.pallas_call(
        partial(_gcn_body, B, F),
        grid=(NI,),
        in_specs=[
            pl.BlockSpec((IB, Y), lambda i: (i, 0)),
            pl.BlockSpec((Y, B * F), lambda i: (0, 0)),
            pl.BlockSpec((IB, F), lambda i: (i, 0)),
            pl.BlockSpec((1, F), lambda i: (0, 0)),
            pl.BlockSpec((B * F, B), lambda i: (0, 0)),
            pl.BlockSpec((B, IB), lambda i: (0, i)),
            pl.BlockSpec((1, IB), lambda i: (0, i)),
        ],
        out_specs=pl.BlockSpec((B, IB), lambda i: (0, i)),
        out_shape=jax.ShapeDtypeStruct((B, Y), jnp.float32),
        compiler_params=pltpu.CompilerParams(
            dimension_semantics=("parallel",),
            vmem_limit_bytes=56 * 1024 * 1024),
        name="gcn_score",
    )(adj, S_flat, wB, gb1, sel, y4a, f4b)

    return y4t, y4
